# baseline (speedup 1.0000x reference)
import jax
import jax.numpy as jnp
from jax.experimental import pallas as pl
from jax.experimental.pallas import tpu as pltpu

B, N, C = 8, 1024, 256
EPS = 1e-5
NLAYERS = 3
PAIR = 2
NPAIRS = B // PAIR
DUMMY = B


def _gcn_kernel(x_ref, adj_ref, W_ref, Wa_ref, b_ref, g_ref, be_ref, out_ref,
                adj_s, h_s, csum_s, sum_s, sq_s):
    l = pl.program_id(0)
    bb = pl.program_id(1)
    f32 = jnp.float32

    def compute_pair(cp, active, zero_stats, do_norm):
        psums, psqs = [], []
        for j in range(PAIR):
            b = PAIR * cp + j
            xin = h_s[b]
            xh = xin.astype(jnp.bfloat16)
            tmp = jnp.dot(xh, Wa_ref[0], preferred_element_type=f32)
            th = tmp.astype(jnp.bfloat16)
            tsum = jnp.dot(csum_s[b], W_ref[0], preferred_element_type=f32)
            corr = 0.5 * tsum + b_ref[0]
            acc = jnp.dot(adj_s[b], th, preferred_element_type=f32) + corr
            h = jnp.maximum(acc, 0.0)
            tgt = jnp.where(active, b, DUMMY)
            h_s[tgt] = h
            ps = jnp.sum(h, axis=0, keepdims=True)
            csum_s[tgt] = ps
            psums.append(ps)
            psqs.append(jnp.sum(h * h, axis=0, keepdims=True))
        live = jnp.where(active, 1.0, 0.0)
        sum_s[...] = (jnp.where(zero_stats, 0.0, sum_s[...])
                      + live * (psums[0] + psums[1]))
        sq_s[...] = (jnp.where(zero_stats, 0.0, sq_s[...])
                     + live * (psqs[0] + psqs[1]))

        @pl.when(do_norm)
        def _():
            cnt = float(B * N)
            mean = sum_s[...] / cnt
            var = sq_s[...] / cnt - mean * mean
            scale = g_ref[0] / jnp.sqrt(var + EPS)
            shift = be_ref[0] - mean * scale

            @pl.when(l < NLAYERS - 1)
            def _():
                h_s[0:B] = h_s[0:B] * scale[None] + shift[None]
                csum_s[0:B] = (csum_s[0:B] * scale[None]
                               + float(N) * shift[None])

            @pl.when(l == NLAYERS - 1)
            def _():
                out_ref[...] = h_s[0:B] * scale[None] + shift[None]

    @pl.when(l == 0)
    def _():
        pb = jnp.minimum(bb, NPAIRS - 1)
        row = jax.lax.broadcasted_iota(jnp.int32, (N, N), 0)
        col = jax.lax.broadcasted_iota(jnp.int32, (N, N), 1)
        fresh = bb <= NPAIRS - 1
        for j in range(PAIR):
            b = PAIR * pb + j
            tgt = jnp.where(fresh, b, DUMMY)
            a32 = adj_ref[j].astype(f32)
            adj_s[jnp.where(fresh, b, B)] = (
                jnp.where(row == col, 1.0, a32) - 0.5).astype(jnp.bfloat16)
            xb = x_ref[j]
            h_s[tgt] = xb
            csum_s[tgt] = jnp.sum(xb, axis=0, keepdims=True)
        cp = jnp.maximum(bb - 1, 0)
        compute_pair(cp, active=bb >= 1, zero_stats=bb == 1,
                     do_norm=bb == NPAIRS)

    @pl.when(jnp.logical_and(l > 0, bb <= NPAIRS - 1))
    def _():
        compute_pair(bb, active=True, zero_stats=bb == 0,
                     do_norm=bb == NPAIRS - 1)


def kernel(x, adj, W1, b1, W2, b2, W3, b3, g1, be1, g2, be2, g3, be3):
    Ws = jnp.stack([W1, W2, W3])
    Was = Ws.astype(jnp.bfloat16)
    bs = jnp.stack([b1, b2, b3])[:, None, :]
    gs = jnp.stack([g1, g2, g3])[:, None, :]
    bes = jnp.stack([be1, be2, be3])[:, None, :]
    adj_bf = adj.astype(jnp.bfloat16)

    pmap = lambda l, bb: (jnp.where(l == 0, jnp.minimum(bb, NPAIRS - 1),
                                    NPAIRS - 1), 0, 0)
    lmap = lambda l, bb: (l, 0, 0)
    return pl.pallas_call(
        _gcn_kernel,
        grid=(NLAYERS, NPAIRS + 1),
        in_specs=[
            pl.BlockSpec((PAIR, N, C), pmap),
            pl.BlockSpec((PAIR, N, N), pmap),
            pl.BlockSpec((1, C, C), lmap),
            pl.BlockSpec((1, C, C), lmap),
            pl.BlockSpec((1, 1, C), lmap),
            pl.BlockSpec((1, 1, C), lmap),
            pl.BlockSpec((1, 1, C), lmap),
        ],
        out_specs=pl.BlockSpec((B, N, C), lambda l, bb: (0, 0, 0)),
        out_shape=jax.ShapeDtypeStruct((B, N, C), jnp.float32),
        scratch_shapes=[
            pltpu.VMEM((B + 1, N, N), jnp.bfloat16),
            pltpu.VMEM((B + 1, N, C), jnp.float32),
            pltpu.VMEM((B + 1, 1, C), jnp.float32),
            pltpu.VMEM((1, C), jnp.float32),
            pltpu.VMEM((1, C), jnp.float32),
        ],
    )(x, adj_bf, Ws, Was, bs, gs, bes)


# speedup vs baseline: 1.3434x; 1.3434x over previous
import jax
import jax.numpy as jnp
from jax.experimental import pallas as pl
from jax.experimental.pallas import tpu as pltpu

B, N, C = 8, 1024, 256
EPS = 1e-5
NLAYERS = 3
PAIR = 2
NPAIRS = B // PAIR


def _gcn_kernel(x_ref, adj_ref, W_ref, Wa_ref, b_ref, g_ref, be_ref, out_ref,
                adj_s, h_s, csum_s, sum_s, sq_s):
    l = pl.program_id(0)
    bb = pl.program_id(1)
    f32 = jnp.float32

    def compute_pair(cp, active, zero_stats, do_norm):
        psums, psqs = [], []
        for j in range(PAIR):
            b = PAIR * cp + j
            xin = h_s[b]
            xh = xin.astype(jnp.bfloat16)
            tmp = jnp.dot(xh, Wa_ref[0], preferred_element_type=f32)
            th = tmp.astype(jnp.bfloat16)
            tsum = jnp.dot(csum_s[b], W_ref[0], preferred_element_type=f32)
            corr = 0.5 * tsum + b_ref[0]
            acc = jnp.dot(adj_s[b], th, preferred_element_type=f32) + corr
            h = jnp.maximum(acc, 0.0)
            tgt = jnp.where(active, b, 2 + j)
            h_s[tgt] = h
            ps = jnp.sum(h, axis=0, keepdims=True)
            csum_s[tgt] = ps
            psums.append(ps)
            psqs.append(jnp.sum(h * h, axis=0, keepdims=True))
        live = jnp.where(active, 1.0, 0.0)
        sum_s[...] = (jnp.where(zero_stats, 0.0, sum_s[...])
                      + live * (psums[0] + psums[1]))
        sq_s[...] = (jnp.where(zero_stats, 0.0, sq_s[...])
                     + live * (psqs[0] + psqs[1]))

        @pl.when(do_norm)
        def _():
            cnt = float(B * N)
            mean = sum_s[...] / cnt
            var = sq_s[...] / cnt - mean * mean
            scale = g_ref[0] / jnp.sqrt(var + EPS)
            shift = be_ref[0] - mean * scale

            @pl.when(l < NLAYERS - 1)
            def _():
                h_s[0:B] = h_s[0:B] * scale[None] + shift[None]
                csum_s[0:B] = (csum_s[0:B] * scale[None]
                               + float(N) * shift[None])

            @pl.when(l == NLAYERS - 1)
            def _():
                out_ref[...] = h_s[0:B] * scale[None] + shift[None]

    @pl.when(l == 0)
    def _():
        pb = jnp.minimum(bb, NPAIRS - 1)
        row = jax.lax.broadcasted_iota(jnp.int32, (N, N), 0)
        col = jax.lax.broadcasted_iota(jnp.int32, (N, N), 1)
        fresh = bb <= NPAIRS - 1
        for j in range(PAIR):
            b = PAIR * pb + j
            adj_s[b] = (jnp.where(row == col, 1.0, adj_ref[j])
                        - 0.5).astype(jnp.bfloat16)
            xb = x_ref[j]
            h_s[b] = xb
            csum_s[b] = jnp.sum(xb, axis=0, keepdims=True)
        cp = jnp.maximum(bb - 1, 0)
        compute_pair(cp, active=bb >= 1, zero_stats=bb == 1,
                     do_norm=bb == NPAIRS)

    @pl.when(jnp.logical_and(l > 0, bb <= NPAIRS - 1))
    def _():
        compute_pair(bb, active=True, zero_stats=bb == 0,
                     do_norm=bb == NPAIRS - 1)


def kernel(x, adj, W1, b1, W2, b2, W3, b3, g1, be1, g2, be2, g3, be3):
    Ws = jnp.stack([W1, W2, W3])
    Was = Ws.astype(jnp.bfloat16)
    bs = jnp.stack([b1, b2, b3])[:, None, :]
    gs = jnp.stack([g1, g2, g3])[:, None, :]
    bes = jnp.stack([be1, be2, be3])[:, None, :]

    pmap = lambda l, bb: (jnp.where(l == 0, jnp.minimum(bb, NPAIRS - 1),
                                    NPAIRS - 1), 0, 0)
    lmap = lambda l, bb: (l, 0, 0)
    return pl.pallas_call(
        _gcn_kernel,
        grid=(NLAYERS, NPAIRS + 1),
        in_specs=[
            pl.BlockSpec((PAIR, N, C), pmap),
            pl.BlockSpec((PAIR, N, N), pmap),
            pl.BlockSpec((1, C, C), lmap),
            pl.BlockSpec((1, C, C), lmap),
            pl.BlockSpec((1, 1, C), lmap),
            pl.BlockSpec((1, 1, C), lmap),
            pl.BlockSpec((1, 1, C), lmap),
        ],
        out_specs=pl.BlockSpec((B, N, C), lambda l, bb: (0, 0, 0)),
        out_shape=jax.ShapeDtypeStruct((B, N, C), jnp.float32),
        scratch_shapes=[
            pltpu.VMEM((B, N, N), jnp.bfloat16),
            pltpu.VMEM((B, N, C), jnp.float32),
            pltpu.VMEM((B, 1, C), jnp.float32),
            pltpu.VMEM((1, C), jnp.float32),
            pltpu.VMEM((1, C), jnp.float32),
        ],
    )(x, adj, Ws, Was, bs, gs, bes)


# speedup vs baseline: 1.6515x; 1.2293x over previous
import jax
import jax.numpy as jnp
from jax.experimental import pallas as pl
from jax.experimental.pallas import tpu as pltpu

B, N, C = 8, 1024, 256
EPS = 1e-5
NLAYERS = 3
PAIR = 2
NPAIRS = B // PAIR


def _gcn_kernel(x_ref, adj_ref, W_ref, Wa_ref, b_ref, g_ref, be_ref, out_ref,
                adj_s, h_s, csum_s, sum_s, sq_s):
    l = pl.program_id(0)
    bb = pl.program_id(1)
    f32 = jnp.float32

    @pl.when(l == 0)
    def _():
        row = jax.lax.broadcasted_iota(jnp.int32, (N, N), 0)
        col = jax.lax.broadcasted_iota(jnp.int32, (N, N), 1)
        for j in range(PAIR):
            b = PAIR * bb + j
            adj_s[b] = (jnp.where(row == col, 1.0, adj_ref[j])
                        - 0.5).astype(jnp.bfloat16)
            xb = x_ref[j]
            h_s[b] = xb
            csum_s[b] = jnp.sum(xb, axis=0, keepdims=True)

    psums = []
    psqs = []
    for j in range(PAIR):
        b = PAIR * bb + j
        xin = h_s[b]
        xh = xin.astype(jnp.bfloat16)
        tmp = jnp.dot(xh, Wa_ref[0], preferred_element_type=f32)
        th = tmp.astype(jnp.bfloat16)
        tsum = jnp.dot(csum_s[b], W_ref[0], preferred_element_type=f32)
        corr = 0.5 * tsum + b_ref[0]
        acc = jnp.dot(adj_s[b], th, preferred_element_type=f32) + corr
        h = jnp.maximum(acc, 0.0)
        h_s[b] = h
        ps = jnp.sum(h, axis=0, keepdims=True)
        csum_s[b] = ps
        psums.append(ps)
        psqs.append(jnp.sum(h * h, axis=0, keepdims=True))

    first = (bb == 0)
    sum_s[...] = jnp.where(first, 0.0, sum_s[...]) + psums[0] + psums[1]
    sq_s[...] = jnp.where(first, 0.0, sq_s[...]) + psqs[0] + psqs[1]

    @pl.when(bb == NPAIRS - 1)
    def _():
        cnt = float(B * N)
        mean = sum_s[...] / cnt
        var = sq_s[...] / cnt - mean * mean
        scale = g_ref[0] / jnp.sqrt(var + EPS)
        shift = be_ref[0] - mean * scale

        @pl.when(l < NLAYERS - 1)
        def _():
            h_s[...] = h_s[...] * scale[None] + shift[None]
            csum_s[...] = csum_s[...] * scale[None] + float(N) * shift[None]

        @pl.when(l == NLAYERS - 1)
        def _():
            out_ref[...] = h_s[...] * scale[None] + shift[None]


def kernel(x, adj, W1, b1, W2, b2, W3, b3, g1, be1, g2, be2, g3, be3):
    Ws = jnp.stack([W1, W2, W3])
    Was = Ws.astype(jnp.bfloat16)
    bs = jnp.stack([b1, b2, b3])[:, None, :]
    gs = jnp.stack([g1, g2, g3])[:, None, :]
    bes = jnp.stack([be1, be2, be3])[:, None, :]

    l0map = lambda l, bb: (jnp.where(l == 0, bb, 0), 0, 0)
    lmap = lambda l, bb: (l, 0, 0)
    return pl.pallas_call(
        _gcn_kernel,
        grid=(NLAYERS, NPAIRS),
        in_specs=[
            pl.BlockSpec((PAIR, N, C), l0map),
            pl.BlockSpec((PAIR, N, N), l0map),
            pl.BlockSpec((1, C, C), lmap),
            pl.BlockSpec((1, C, C), lmap),
            pl.BlockSpec((1, 1, C), lmap),
            pl.BlockSpec((1, 1, C), lmap),
            pl.BlockSpec((1, 1, C), lmap),
        ],
        out_specs=pl.BlockSpec((B, N, C), lambda l, bb: (0, 0, 0)),
        out_shape=jax.ShapeDtypeStruct((B, N, C), jnp.float32),
        scratch_shapes=[
            pltpu.VMEM((B, N, N), jnp.bfloat16),
            pltpu.VMEM((B, N, C), jnp.float32),
            pltpu.VMEM((B, 1, C), jnp.float32),
            pltpu.VMEM((1, C), jnp.float32),
            pltpu.VMEM((1, C), jnp.float32),
        ],
    )(x, adj, Ws, Was, bs, gs, bes)
